# baseline (speedup 1.0000x reference)
import functools

import jax
import jax.numpy as jnp
from jax import lax
from jax.experimental import pallas as pl
from jax.experimental.pallas import tpu as pltpu
from jax.experimental.pallas import tpu_sc as plsc

_MAX_SIZE = 4096
_N_PROTO = 51
_FEAT_DIM = 128
_N_REL = 16384
_INV_THRESH = 1.0 / 0.9

_RB = 2048
_NRB = _N_REL // _RB

_TCH = 1024
_NTCH = _N_REL // _TCH


def _route_tc_body(logits_ref, slot_ref, targ_ref, cnt_ref):
  z = logits_ref[...]
  mx = jnp.max(z, axis=-1, keepdims=True)
  lane = lax.broadcasted_iota(jnp.int32, z.shape, 1)
  am = jnp.min(jnp.where(z >= mx, lane, _N_PROTO), axis=-1)
  ssum = jnp.sum(jnp.exp(z - mx), axis=-1)
  conf = ssum < _INV_THRESH
  targ = am * _MAX_SIZE + slot_ref[...]
  targ_ref[...] = jnp.where(conf, targ, -1)
  cnt_ref[0, 0, 0] = jnp.sum(jnp.where(conf, 1, 0)).astype(jnp.int32)


_route = pl.pallas_call(
    _route_tc_body,
    grid=(_NRB,),
    in_specs=[
        pl.BlockSpec((_RB, _N_PROTO), lambda i: (i, 0)),
        pl.BlockSpec((_RB,), lambda i: (i,)),
    ],
    out_specs=[
        pl.BlockSpec((_RB,), lambda i: (i,)),
        pl.BlockSpec((1, 1, 1), lambda i: (i, 0, 0), memory_space=pltpu.SMEM),
    ],
    out_shape=[
        jax.ShapeDtypeStruct((_N_REL,), jnp.int32),
        jax.ShapeDtypeStruct((_NRB, 1, 1), jnp.int32),
    ],
    name="memory_banks_route",
)


def _scs_body(feature_hbm, targ_hbm, cnt_hbm, mem_ref,
              cnt_s, targ_s, row_v):
  pltpu.sync_copy(cnt_hbm, cnt_s)
  cnt = cnt_s[0, 0, 0]
  for b in range(1, _NRB):
    cnt = cnt + cnt_s[b, 0, 0]

  @pl.when(cnt > 0)
  def _rare():
    def chunk(k, carry):
      pltpu.sync_copy(targ_hbm.at[pl.ds(k * _TCH, _TCH)], targ_s)

      def cand(i, c2):
        t = targ_s[i]

        @pl.when(t >= 0)
        def _write():
          pltpu.sync_copy(feature_hbm.at[pl.ds(k * _TCH + i, 1), :], row_v)
          pltpu.sync_copy(row_v, mem_ref.at[pl.ds(t, 1), :])

        return c2

      lax.fori_loop(0, _TCH, cand, 0)
      return carry

    lax.fori_loop(0, _NTCH, chunk, 0)


_smesh = plsc.ScalarSubcoreMesh(axis_name="c", num_cores=1)

_scatter = pl.kernel(
    _scs_body,
    out_type=(),
    mesh=_smesh,
    scratch_types=[
        pltpu.SMEM((_NRB, 1, 1), jnp.int32),
        pltpu.SMEM((_TCH,), jnp.int32),
        pltpu.VMEM_SHARED((1, _FEAT_DIM), jnp.float32),
    ],
    name="memory_banks_scatter",
)


def kernel(mem, feature, rel_logits, slot_idx):
  targ_enc, counts = _route(rel_logits, slot_idx)
  mem_ref = jax.new_ref(mem)
  _scatter(feature, targ_enc, counts, mem_ref)
  return mem_ref[...]


# speedup vs baseline: 1.2210x; 1.2210x over previous
import functools

import jax
import jax.numpy as jnp
from jax import lax
from jax.experimental import pallas as pl
from jax.experimental.pallas import tpu as pltpu
from jax.experimental.pallas import tpu_sc as plsc

_MAX_SIZE = 4096
_N_PROTO = 51
_FEAT_DIM = 128
_N_REL = 16384
_INV_THRESH = 1.0 / 0.9

_RB = 2048
_NRB = _N_REL // _RB

_TCH = 1024
_NTCH = _N_REL // _TCH


def _route_tc_body(logits_ref, slot_ref, targ_ref, cnt_ref):
  z = logits_ref[...]
  mx = jnp.max(z, axis=0, keepdims=True)
  cls = lax.broadcasted_iota(jnp.int32, z.shape, 0)
  am = jnp.min(jnp.where(z >= mx, cls, _N_PROTO), axis=0)
  ssum = jnp.sum(jnp.exp(z - mx), axis=0)
  conf = ssum < _INV_THRESH
  targ = am * _MAX_SIZE + slot_ref[...]
  targ_ref[...] = jnp.where(conf, targ, -1)
  cnt_ref[0, 0, 0] = jnp.sum(jnp.where(conf, 1, 0)).astype(jnp.int32)


_route = pl.pallas_call(
    _route_tc_body,
    grid=(_NRB,),
    in_specs=[
        pl.BlockSpec((_N_PROTO, _RB), lambda i: (0, i)),
        pl.BlockSpec((_RB,), lambda i: (i,)),
    ],
    out_specs=[
        pl.BlockSpec((_RB,), lambda i: (i,)),
        pl.BlockSpec((1, 1, 1), lambda i: (i, 0, 0), memory_space=pltpu.SMEM),
    ],
    out_shape=[
        jax.ShapeDtypeStruct((_N_REL,), jnp.int32),
        jax.ShapeDtypeStruct((_NRB, 1, 1), jnp.int32),
    ],
    name="memory_banks_route",
)


def _scs_body(feature_hbm, targ_hbm, cnt_hbm, mem_ref,
              cnt_s, targ_s, row_v):
  pltpu.sync_copy(cnt_hbm, cnt_s)
  cnt = cnt_s[0, 0, 0]
  for b in range(1, _NRB):
    cnt = cnt + cnt_s[b, 0, 0]

  @pl.when(cnt > 0)
  def _rare():
    def chunk(k, carry):
      pltpu.sync_copy(targ_hbm.at[pl.ds(k * _TCH, _TCH)], targ_s)

      def cand(i, c2):
        t = targ_s[i]

        @pl.when(t >= 0)
        def _write():
          pltpu.sync_copy(feature_hbm.at[pl.ds(k * _TCH + i, 1), :], row_v)
          pltpu.sync_copy(row_v, mem_ref.at[pl.ds(t, 1), :])

        return c2

      lax.fori_loop(0, _TCH, cand, 0)
      return carry

    lax.fori_loop(0, _NTCH, chunk, 0)


_smesh = plsc.ScalarSubcoreMesh(axis_name="c", num_cores=1)

_scatter = pl.kernel(
    _scs_body,
    out_type=(),
    mesh=_smesh,
    scratch_types=[
        pltpu.SMEM((_NRB, 1, 1), jnp.int32),
        pltpu.SMEM((_TCH,), jnp.int32),
        pltpu.VMEM_SHARED((1, _FEAT_DIM), jnp.float32),
    ],
    name="memory_banks_scatter",
)


def kernel(mem, feature, rel_logits, slot_idx):
  logits_t = rel_logits.T
  targ_enc, counts = _route(logits_t, slot_idx)
  mem_ref = jax.new_ref(mem)
  _scatter(feature, targ_enc, counts, mem_ref)
  return mem_ref[...]


# speedup vs baseline: 1.2482x; 1.0223x over previous
import functools

import jax
import jax.numpy as jnp
from jax import lax
from jax.experimental import pallas as pl
from jax.experimental.pallas import tpu as pltpu
from jax.experimental.pallas import tpu_sc as plsc

_MAX_SIZE = 4096
_N_PROTO = 51
_FEAT_DIM = 128
_N_REL = 16384
_INV_THRESH = 1.0 / 0.9

_RB = 4096
_NRB = _N_REL // _RB

_TCH = 1024
_NTCH = _N_REL // _TCH


def _route_tc_body(logits_ref, slot_ref, targ_ref, cnt_ref):
  z = logits_ref[...]
  mx = jnp.max(z, axis=0, keepdims=True)
  cls = lax.broadcasted_iota(jnp.int32, z.shape, 0)
  am = jnp.min(jnp.where(z >= mx, cls, _N_PROTO), axis=0)
  ssum = jnp.sum(jnp.exp(z - mx), axis=0)
  conf = ssum < _INV_THRESH
  targ = am * _MAX_SIZE + slot_ref[...]
  targ_ref[...] = jnp.where(conf, targ, -1)
  cnt_ref[0, 0, 0] = jnp.sum(jnp.where(conf, 1, 0)).astype(jnp.int32)


_route = pl.pallas_call(
    _route_tc_body,
    grid=(_NRB,),
    in_specs=[
        pl.BlockSpec((_N_PROTO, _RB), lambda i: (0, i)),
        pl.BlockSpec((_RB,), lambda i: (i,)),
    ],
    out_specs=[
        pl.BlockSpec((_RB,), lambda i: (i,)),
        pl.BlockSpec((1, 1, 1), lambda i: (i, 0, 0), memory_space=pltpu.SMEM),
    ],
    out_shape=[
        jax.ShapeDtypeStruct((_N_REL,), jnp.int32),
        jax.ShapeDtypeStruct((_NRB, 1, 1), jnp.int32),
    ],
    name="memory_banks_route",
)


def _scs_body(feature_hbm, targ_hbm, cnt_hbm, mem_ref,
              cnt_s, targ_s, row_v):
  pltpu.sync_copy(cnt_hbm, cnt_s)
  cnt = cnt_s[0, 0, 0]
  for b in range(1, _NRB):
    cnt = cnt + cnt_s[b, 0, 0]

  @pl.when(cnt > 0)
  def _rare():
    def chunk(k, carry):
      pltpu.sync_copy(targ_hbm.at[pl.ds(k * _TCH, _TCH)], targ_s)

      def cand(i, c2):
        t = targ_s[i]

        @pl.when(t >= 0)
        def _write():
          pltpu.sync_copy(feature_hbm.at[pl.ds(k * _TCH + i, 1), :], row_v)
          pltpu.sync_copy(row_v, mem_ref.at[pl.ds(t, 1), :])

        return c2

      lax.fori_loop(0, _TCH, cand, 0)
      return carry

    lax.fori_loop(0, _NTCH, chunk, 0)


_smesh = plsc.ScalarSubcoreMesh(axis_name="c", num_cores=1)

_scatter = pl.kernel(
    _scs_body,
    out_type=(),
    mesh=_smesh,
    scratch_types=[
        pltpu.SMEM((_NRB, 1, 1), jnp.int32),
        pltpu.SMEM((_TCH,), jnp.int32),
        pltpu.VMEM_SHARED((1, _FEAT_DIM), jnp.float32),
    ],
    name="memory_banks_scatter",
)


def kernel(mem, feature, rel_logits, slot_idx):
  logits_t = rel_logits.T
  targ_enc, counts = _route(logits_t, slot_idx)
  mem_ref = jax.new_ref(mem)
  _scatter(feature, targ_enc, counts, mem_ref)
  return mem_ref[...]
